# baseline (speedup 1.0000x reference)
import functools

import jax
import jax.numpy as jnp
from jax import lax
from jax.experimental import pallas as pl
from jax.experimental.pallas import tpu as pltpu
from jax.experimental.pallas import tpu_sc as plsc

NUM_SEG = 201
SEG_PAD = 208
V_TOTAL = 46 * 55 * 46
CHANNELS = 256
NB, NCH = 4, 64
NW = 32
BLK = 48
NBLK_FULL = V_TOTAL // BLK
TAIL_ROWS = V_TOTAL - NBLK_FULL * BLK
KFULL = 75
REMW = NBLK_FULL - KFULL * NW
ACC_WORDS = SEG_PAD * CHANNELS

_GDN = lax.GatherDimensionNumbers(
    offset_dims=(), collapsed_slice_dims=(0,), start_index_map=(0,))


def _splat(vec, u):
    return lax.gather(vec, jnp.full((16, 1), u, jnp.int32), _GDN, (1,),
                      mode=lax.GatherScatterMode.PROMISE_IN_BOUNDS)


def _seg_sum_kernel(data_hbm, labels_hbm, out_hbm, cnt_hbm,
                    lab0, lab1, buf0, buf1, lab_t, buf_t, acc_v, cnt_v,
                    sl0, sd0, sl1, sd1, st):
    cid = lax.axis_index("c")
    sid = lax.axis_index("s")
    wid = sid * 2 + cid

    labs = [lab0, lab1]
    bufs = [buf0, buf1]
    sems = [(sl0, sd0), (sl1, sd1)]

    zero16 = jnp.zeros((16,), jnp.float32)
    ones16 = jnp.ones((16,), jnp.float32)
    cvecs = [jnp.arange(16, dtype=jnp.int32) + g * 16 for g in range(16)]

    def zbody(z, _):
        acc_v[pl.ds(z * 16, 16)] = zero16
        return 0
    lax.fori_loop(0, ACC_WORDS // 16, zbody, 0)
    for r in range(SEG_PAD // 16):
        cnt_v[pl.ds(r * 16, 16)] = zero16

    def issue(slot, g):
        slab, sdat = sems[slot]
        pltpu.async_copy(labels_hbm.at[pl.ds(g * BLK, BLK)], labs[slot], slab)
        pltpu.async_copy(data_hbm.at[pl.ds(g * BLK, BLK), :, :], bufs[slot],
                         sdat)

    def drain(slot):
        slab, sdat = sems[slot]
        pltpu.make_async_copy(labels_hbm.at[pl.ds(0, BLK)],
                              labs[slot], slab).wait()
        pltpu.make_async_copy(data_hbm.at[pl.ds(0, BLK), :, :],
                              bufs[slot], sdat).wait()

    def process(slot):
        lab_v = labs[slot]
        buf = bufs[slot]

        def vgroup(j, _):
            lv = lab_v[pl.ds(j * 16, 16)]
            plsc.addupdate_scatter(cnt_v, [lv], ones16)
            for u in range(16):
                v = j * 16 + u
                lbase = _splat(lv, u) << 8
                for b in range(NB):
                    for g in range(NCH // 16):
                        vals = buf[v, b, pl.ds(g * 16, 16)]
                        plsc.addupdate_scatter(
                            acc_v, [lbase + cvecs[b * 4 + g]], vals)
            return 0
        lax.fori_loop(0, BLK // 16, vgroup, 0)

    issue(0, wid)

    def ring(i, _):
        issue(1, (2 * i + 1) * NW + wid)
        drain(0)
        process(0)
        issue(0, (2 * i + 2) * NW + wid)
        drain(1)
        process(1)
        return 0
    lax.fori_loop(0, (KFULL - 1) // 2, ring, 0)

    drain(0)
    process(0)

    @pl.when(wid < REMW)
    def _():
        issue(0, KFULL * NW + wid)
        drain(0)
        process(0)

    @pl.when(wid == REMW)
    def _():
        off = NBLK_FULL * BLK
        pltpu.async_copy(labels_hbm.at[pl.ds(off, TAIL_ROWS)], lab_t, st)
        pltpu.make_async_copy(labels_hbm.at[pl.ds(0, TAIL_ROWS)],
                              lab_t, st).wait()
        pltpu.async_copy(data_hbm.at[pl.ds(off, TAIL_ROWS), :, :], buf_t, st)
        pltpu.make_async_copy(data_hbm.at[pl.ds(0, TAIL_ROWS), :, :],
                              buf_t, st).wait()
        for j in range(2):
            base = j * 12
            lv = lab_t[pl.ds(base, 16)]
            mask = jnp.arange(16, dtype=jnp.int32) >= (4 * j)
            plsc.addupdate_scatter(cnt_v, [lv], ones16, mask=mask)
        def tvox(v, _):
            vsp = jnp.full((16,), v, jnp.int32)
            lbase = plsc.load_gather(lab_t, [vsp]) << 8
            for b in range(NB):
                for g in range(NCH // 16):
                    vals = buf_t[v, b, pl.ds(g * 16, 16)]
                    plsc.addupdate_scatter(
                        acc_v, [lbase + cvecs[b * 4 + g]], vals)
            return 0
        lax.fori_loop(0, TAIL_ROWS, tvox, 0)

    pltpu.sync_copy(acc_v, out_hbm.at[pl.ds(wid * ACC_WORDS, ACC_WORDS)])
    pltpu.sync_copy(cnt_v, cnt_hbm.at[pl.ds(wid * SEG_PAD, SEG_PAD)])


_seg_sum = functools.partial(
    pl.kernel,
    out_type=[
        jax.ShapeDtypeStruct((NW * ACC_WORDS,), jnp.float32),
        jax.ShapeDtypeStruct((NW * SEG_PAD,), jnp.float32),
    ],
    mesh=plsc.VectorSubcoreMesh(core_axis_name="c", subcore_axis_name="s"),
    compiler_params=pltpu.CompilerParams(
        needs_layout_passes=False, use_tc_tiling_on_sc=True),
    scratch_types=[
        pltpu.VMEM((BLK,), jnp.int32),
        pltpu.VMEM((BLK,), jnp.int32),
        pltpu.VMEM((BLK, NB, NCH), jnp.float32),
        pltpu.VMEM((BLK, NB, NCH), jnp.float32),
        pltpu.VMEM((TAIL_ROWS,), jnp.int32),
        pltpu.VMEM((TAIL_ROWS, NB, NCH), jnp.float32),
        pltpu.VMEM((ACC_WORDS,), jnp.float32),
        pltpu.VMEM((SEG_PAD,), jnp.float32),
    ] + [pltpu.SemaphoreType.DMA] * 5,
)(_seg_sum_kernel)


def kernel(feature_map, atlas_labels):
    B, C, D, H, W = feature_map.shape
    V = D * H * W
    dataT = feature_map.transpose(2, 3, 4, 0, 1).reshape(V, B, C)
    labels = atlas_labels.reshape(-1).astype(jnp.int32)
    parts, pcnts = _seg_sum(dataT, labels)
    sums = parts.reshape(NW, SEG_PAD, NB, NCH).sum(0)
    counts = pcnts.reshape(NW, SEG_PAD).sum(0)
    cn = counts[:, None, None]
    means = jnp.where(cn > 0, sums / jnp.maximum(cn, 1.0), 0.0)
    roi = means[1:NUM_SEG]
    roi_features = roi.transpose(1, 0, 2)
    valid = counts[1:NUM_SEG] > 0
    roi_valid_mask = jnp.broadcast_to(valid[None, :], (B, NUM_SEG - 1))
    return (roi_features, roi_valid_mask)


# speedup vs baseline: 1.0635x; 1.0635x over previous
import functools

import jax
import jax.numpy as jnp
from jax import lax
from jax.experimental import pallas as pl
from jax.experimental.pallas import tpu as pltpu
from jax.experimental.pallas import tpu_sc as plsc

NUM_SEG = 201
SEG_PAD = 208
V_TOTAL = 46 * 55 * 46
CHANNELS = 256
NB, NCH = 4, 64
NW = 32
BLK = 48
NBLK_FULL = V_TOTAL // BLK
TAIL_ROWS = V_TOTAL - NBLK_FULL * BLK
KFULL = 75
REMW = NBLK_FULL - KFULL * NW
ACC_WORDS = SEG_PAD * CHANNELS
ACC_B = SEG_PAD * NCH

_GDN = lax.GatherDimensionNumbers(
    offset_dims=(), collapsed_slice_dims=(0,), start_index_map=(0,))


def _splat(vec, u):
    return lax.gather(vec, jnp.full((16, 1), u, jnp.int32), _GDN, (1,),
                      mode=lax.GatherScatterMode.PROMISE_IN_BOUNDS)


def _seg_sum_kernel(data_hbm, labels_hbm, out_hbm, cnt_hbm,
                    lab0, lab1, buf0, buf1, lab_t, buf_t,
                    acc0, acc1, acc2, acc3, cnt_v,
                    sl0, sd0, sl1, sd1, st):
    cid = lax.axis_index("c")
    sid = lax.axis_index("s")
    wid = sid * 2 + cid

    labs = [lab0, lab1]
    bufs = [buf0, buf1]
    accs = [acc0, acc1, acc2, acc3]
    sems = [(sl0, sd0), (sl1, sd1)]

    zero16 = jnp.zeros((16,), jnp.float32)
    ones16 = jnp.ones((16,), jnp.float32)
    cvecs = [jnp.arange(16, dtype=jnp.int32) + g * 16 for g in range(4)]

    def zbody(z, _):
        for acc in accs:
            acc[pl.ds(z * 16, 16)] = zero16
        return 0
    lax.fori_loop(0, ACC_B // 16, zbody, 0)
    for r in range(SEG_PAD // 16):
        cnt_v[pl.ds(r * 16, 16)] = zero16

    def issue(slot, g):
        slab, sdat = sems[slot]
        pltpu.async_copy(labels_hbm.at[pl.ds(g * BLK, BLK)], labs[slot], slab)
        pltpu.async_copy(data_hbm.at[pl.ds(g * BLK, BLK), :, :], bufs[slot],
                         sdat)

    def drain(slot):
        slab, sdat = sems[slot]
        pltpu.make_async_copy(labels_hbm.at[pl.ds(0, BLK)],
                              labs[slot], slab).wait()
        pltpu.make_async_copy(data_hbm.at[pl.ds(0, BLK), :, :],
                              bufs[slot], sdat).wait()

    def process(slot):
        lab_v = labs[slot]
        buf = bufs[slot]

        def vgroup(j, _):
            lv = lab_v[pl.ds(j * 16, 16)]
            plsc.addupdate_scatter(cnt_v, [lv], ones16)
            for u in range(16):
                v = j * 16 + u
                lbase = _splat(lv, u) << 6
                for g in range(NCH // 16):
                    idx = lbase + cvecs[g]
                    for b in range(NB):
                        vals = buf[v, b, pl.ds(g * 16, 16)]
                        plsc.addupdate_scatter(accs[b], [idx], vals)
            return 0
        lax.fori_loop(0, BLK // 16, vgroup, 0)

    issue(0, wid)

    def ring(i, _):
        issue(1, (2 * i + 1) * NW + wid)
        drain(0)
        process(0)
        issue(0, (2 * i + 2) * NW + wid)
        drain(1)
        process(1)
        return 0
    lax.fori_loop(0, (KFULL - 1) // 2, ring, 0)

    drain(0)
    process(0)

    @pl.when(wid < REMW)
    def _():
        issue(0, KFULL * NW + wid)
        drain(0)
        process(0)

    @pl.when(wid == REMW)
    def _():
        off = NBLK_FULL * BLK
        pltpu.async_copy(labels_hbm.at[pl.ds(off, TAIL_ROWS)], lab_t, st)
        pltpu.make_async_copy(labels_hbm.at[pl.ds(0, TAIL_ROWS)],
                              lab_t, st).wait()
        pltpu.async_copy(data_hbm.at[pl.ds(off, TAIL_ROWS), :, :], buf_t, st)
        pltpu.make_async_copy(data_hbm.at[pl.ds(0, TAIL_ROWS), :, :],
                              buf_t, st).wait()
        for j in range(2):
            base = j * 12
            lv = lab_t[pl.ds(base, 16)]
            mask = jnp.arange(16, dtype=jnp.int32) >= (4 * j)
            plsc.addupdate_scatter(cnt_v, [lv], ones16, mask=mask)
        def tvox(v, _):
            vsp = jnp.full((16,), v, jnp.int32)
            lbase = plsc.load_gather(lab_t, [vsp]) << 6
            for g in range(NCH // 16):
                idx = lbase + cvecs[g]
                for b in range(NB):
                    vals = buf_t[v, b, pl.ds(g * 16, 16)]
                    plsc.addupdate_scatter(accs[b], [idx], vals)
            return 0
        lax.fori_loop(0, TAIL_ROWS, tvox, 0)

    for b in range(NB):
        pltpu.sync_copy(accs[b],
                        out_hbm.at[pl.ds(wid * ACC_WORDS + b * ACC_B, ACC_B)])
    pltpu.sync_copy(cnt_v, cnt_hbm.at[pl.ds(wid * SEG_PAD, SEG_PAD)])


_seg_sum = functools.partial(
    pl.kernel,
    out_type=[
        jax.ShapeDtypeStruct((NW * ACC_WORDS,), jnp.float32),
        jax.ShapeDtypeStruct((NW * SEG_PAD,), jnp.float32),
    ],
    mesh=plsc.VectorSubcoreMesh(core_axis_name="c", subcore_axis_name="s"),
    compiler_params=pltpu.CompilerParams(
        needs_layout_passes=False, use_tc_tiling_on_sc=True),
    scratch_types=[
        pltpu.VMEM((BLK,), jnp.int32),
        pltpu.VMEM((BLK,), jnp.int32),
        pltpu.VMEM((BLK, NB, NCH), jnp.float32),
        pltpu.VMEM((BLK, NB, NCH), jnp.float32),
        pltpu.VMEM((TAIL_ROWS,), jnp.int32),
        pltpu.VMEM((TAIL_ROWS, NB, NCH), jnp.float32),
        pltpu.VMEM((ACC_B,), jnp.float32),
        pltpu.VMEM((ACC_B,), jnp.float32),
        pltpu.VMEM((ACC_B,), jnp.float32),
        pltpu.VMEM((ACC_B,), jnp.float32),
        pltpu.VMEM((SEG_PAD,), jnp.float32),
    ] + [pltpu.SemaphoreType.DMA] * 5,
)(_seg_sum_kernel)


def kernel(feature_map, atlas_labels):
    B, C, D, H, W = feature_map.shape
    V = D * H * W
    dataT = feature_map.transpose(2, 3, 4, 0, 1).reshape(V, B, C)
    labels = atlas_labels.reshape(-1).astype(jnp.int32)
    parts, pcnts = _seg_sum(dataT, labels)
    sums = parts.reshape(NW, NB, SEG_PAD, NCH).sum(0)
    sums = sums.transpose(1, 0, 2)
    counts = pcnts.reshape(NW, SEG_PAD).sum(0)
    cn = counts[:, None, None]
    means = jnp.where(cn > 0, sums / jnp.maximum(cn, 1.0), 0.0)
    roi = means[1:NUM_SEG]
    roi_features = roi.transpose(1, 0, 2)
    valid = counts[1:NUM_SEG] > 0
    roi_valid_mask = jnp.broadcast_to(valid[None, :], (B, NUM_SEG - 1))
    return (roi_features, roi_valid_mask)


# speedup vs baseline: 1.7749x; 1.6689x over previous
import functools

import jax
import jax.numpy as jnp
from jax import lax
from jax.experimental import pallas as pl
from jax.experimental.pallas import tpu as pltpu
from jax.experimental.pallas import tpu_sc as plsc

NUM_SEG = 201
SEG_PAD = 208
V_TOTAL = 46 * 55 * 46
CHANNELS = 256
NB, NCH = 4, 64
NW = 32
BLK = 48
NBLK_FULL = V_TOTAL // BLK
TAIL_ROWS = V_TOTAL - NBLK_FULL * BLK
KFULL = 75
REMW = NBLK_FULL - KFULL * NW
ACC_WORDS = SEG_PAD * CHANNELS
ACC_B = SEG_PAD * NCH

_GDN = lax.GatherDimensionNumbers(
    offset_dims=(), collapsed_slice_dims=(0,), start_index_map=(0,))


def _splat(vec, u):
    return lax.gather(vec, jnp.full((16, 1), u, jnp.int32), _GDN, (1,),
                      mode=lax.GatherScatterMode.PROMISE_IN_BOUNDS)


def _seg_sum_kernel(data_hbm, labels_hbm, out_hbm, cnt_hbm,
                    lab0, lab1, buf0, buf1, lab_t, buf_t,
                    acc0, acc1, acc2, acc3, cnt_v,
                    sl0, sd0, sl1, sd1, st):
    cid = lax.axis_index("c")
    sid = lax.axis_index("s")
    wid = sid * 2 + cid

    labs = [lab0, lab1]
    bufs = [buf0, buf1]
    accs = [acc0, acc1, acc2, acc3]
    sems = [(sl0, sd0), (sl1, sd1)]

    zero16 = jnp.zeros((16,), jnp.float32)
    ones16 = jnp.ones((16,), jnp.float32)
    cvecs = [jnp.arange(16, dtype=jnp.int32) + g * 16 for g in range(4)]

    def zbody(z, _):
        for acc in accs:
            acc[pl.ds(z * 16, 16)] = zero16
        return 0
    lax.fori_loop(0, ACC_B // 16, zbody, 0)
    for r in range(SEG_PAD // 16):
        cnt_v[pl.ds(r * 16, 16)] = zero16

    def issue(slot, g):
        slab, sdat = sems[slot]
        pltpu.async_copy(labels_hbm.at[pl.ds(g * BLK, BLK)], labs[slot], slab)
        pltpu.async_copy(data_hbm.at[pl.ds(g * BLK, BLK), :, :], bufs[slot],
                         sdat)

    def drain(slot):
        slab, sdat = sems[slot]
        pltpu.make_async_copy(labels_hbm.at[pl.ds(0, BLK)],
                              labs[slot], slab).wait()
        pltpu.make_async_copy(data_hbm.at[pl.ds(0, BLK), :, :],
                              bufs[slot], sdat).wait()

    def process(slot):
        lab_v = labs[slot]
        buf = bufs[slot]

        def vgroup(j, _):
            lv = lab_v[pl.ds(j * 16, 16)]
            plsc.addupdate_scatter(cnt_v, [lv], ones16)
            for u in range(16):
                v = j * 16 + u
                lbase = _splat(lv, u) << 6
                vals = [[buf[v, b, pl.ds(g * 16, 16)] for b in range(NB)]
                        for g in range(NCH // 16)]
                for g in range(NCH // 16):
                    idx = lbase + cvecs[g]
                    for b in range(NB):
                        plsc.addupdate_scatter(accs[b], [idx], vals[g][b])
            return 0
        lax.fori_loop(0, BLK // 16, vgroup, 0)

    issue(0, wid)

    def ring(i, _):
        issue(1, (2 * i + 1) * NW + wid)
        drain(0)
        process(0)
        issue(0, (2 * i + 2) * NW + wid)
        drain(1)
        process(1)
        return 0
    lax.fori_loop(0, (KFULL - 1) // 2, ring, 0)

    drain(0)
    process(0)

    @pl.when(wid < REMW)
    def _():
        issue(0, KFULL * NW + wid)
        drain(0)
        process(0)

    @pl.when(wid == REMW)
    def _():
        off = NBLK_FULL * BLK
        pltpu.async_copy(labels_hbm.at[pl.ds(off, TAIL_ROWS)], lab_t, st)
        pltpu.make_async_copy(labels_hbm.at[pl.ds(0, TAIL_ROWS)],
                              lab_t, st).wait()
        pltpu.async_copy(data_hbm.at[pl.ds(off, TAIL_ROWS), :, :], buf_t, st)
        pltpu.make_async_copy(data_hbm.at[pl.ds(0, TAIL_ROWS), :, :],
                              buf_t, st).wait()
        for j in range(2):
            base = j * 12
            lv = lab_t[pl.ds(base, 16)]
            mask = jnp.arange(16, dtype=jnp.int32) >= (4 * j)
            plsc.addupdate_scatter(cnt_v, [lv], ones16, mask=mask)
        def tvox(v, _):
            vsp = jnp.full((16,), v, jnp.int32)
            lbase = plsc.load_gather(lab_t, [vsp]) << 6
            for g in range(NCH // 16):
                idx = lbase + cvecs[g]
                for b in range(NB):
                    vals = buf_t[v, b, pl.ds(g * 16, 16)]
                    plsc.addupdate_scatter(accs[b], [idx], vals)
            return 0
        lax.fori_loop(0, TAIL_ROWS, tvox, 0)

    for b in range(NB):
        pltpu.sync_copy(accs[b],
                        out_hbm.at[pl.ds(wid * ACC_WORDS + b * ACC_B, ACC_B)])
    pltpu.sync_copy(cnt_v, cnt_hbm.at[pl.ds(wid * SEG_PAD, SEG_PAD)])


_seg_sum = functools.partial(
    pl.kernel,
    out_type=[
        jax.ShapeDtypeStruct((NW * ACC_WORDS,), jnp.float32),
        jax.ShapeDtypeStruct((NW * SEG_PAD,), jnp.float32),
    ],
    mesh=plsc.VectorSubcoreMesh(core_axis_name="c", subcore_axis_name="s"),
    compiler_params=pltpu.CompilerParams(
        needs_layout_passes=False, use_tc_tiling_on_sc=True),
    scratch_types=[
        pltpu.VMEM((BLK,), jnp.int32),
        pltpu.VMEM((BLK,), jnp.int32),
        pltpu.VMEM((BLK, NB, NCH), jnp.float32),
        pltpu.VMEM((BLK, NB, NCH), jnp.float32),
        pltpu.VMEM((TAIL_ROWS,), jnp.int32),
        pltpu.VMEM((TAIL_ROWS, NB, NCH), jnp.float32),
        pltpu.VMEM((ACC_B,), jnp.float32),
        pltpu.VMEM((ACC_B,), jnp.float32),
        pltpu.VMEM((ACC_B,), jnp.float32),
        pltpu.VMEM((ACC_B,), jnp.float32),
        pltpu.VMEM((SEG_PAD,), jnp.float32),
    ] + [pltpu.SemaphoreType.DMA] * 5,
)(_seg_sum_kernel)


def kernel(feature_map, atlas_labels):
    B, C, D, H, W = feature_map.shape
    V = D * H * W
    dataT = feature_map.transpose(2, 3, 4, 0, 1).reshape(V, B, C)
    labels = atlas_labels.reshape(-1).astype(jnp.int32)
    parts, pcnts = _seg_sum(dataT, labels)
    sums = parts.reshape(NW, NB, SEG_PAD, NCH).sum(0)
    sums = sums.transpose(1, 0, 2)
    counts = pcnts.reshape(NW, SEG_PAD).sum(0)
    cn = counts[:, None, None]
    means = jnp.where(cn > 0, sums / jnp.maximum(cn, 1.0), 0.0)
    roi = means[1:NUM_SEG]
    roi_features = roi.transpose(1, 0, 2)
    valid = counts[1:NUM_SEG] > 0
    roi_valid_mask = jnp.broadcast_to(valid[None, :], (B, NUM_SEG - 1))
    return (roi_features, roi_valid_mask)
